# Initial kernel scaffold; baseline (speedup 1.0000x reference)
import jax
import jax.numpy as jnp
from jax.experimental import pallas as pl


def kernel(x, xyz, alpha, beta):
    raise NotImplementedError("write your pallas kernel here")


# speedup vs baseline: 19.4081x; 19.4081x over previous
import jax
import jax.numpy as jnp
from jax.experimental import pallas as pl
from jax.experimental.pallas import tpu as pltpu

_K = 24
_QBLK = 256


def _ga_block_kernel(xyz_ref, x_ref, alpha_ref, beta_ref, out_ref, p_ref):
    qi = pl.program_id(1)
    n = x_ref.shape[2]
    c = x_ref.shape[1]

    keys = xyz_ref[0]
    q = xyz_ref[0, :, pl.ds(qi * _QBLK, _QBLK)]
    ksq = jnp.sum(keys * keys, axis=0, keepdims=True)
    cross = jax.lax.dot_general(
        q, keys, (((0,), (0,)), ((), ())),
        preferred_element_type=jnp.float32)
    p_ref[...] = -2.0 * cross + ksq

    def body(_, carry):
        p = p_ref[...]
        m = jnp.min(p, axis=1, keepdims=True)
        p_ref[...] = jnp.where(p == m, jnp.inf, p)
        return carry

    jax.lax.fori_loop(0, _K, body, 0)
    sel = (p_ref[...] == jnp.inf).astype(jnp.float32)

    x_b = x_ref[0]
    xa = jnp.concatenate(
        [x_b, x_b * x_b, jnp.ones((1, n), jnp.float32)], axis=0)
    s = jax.lax.dot_general(
        xa, sel, (((1,), (1,)), ((), ())),
        preferred_element_type=jnp.float32,
        precision=jax.lax.Precision.HIGHEST)
    s1 = s[:c]
    s2 = s[c:2 * c]
    cnt = s[2 * c:2 * c + 1]
    mean = s1 / cnt
    var = jnp.maximum((s2 - cnt * mean * mean) / (cnt - 1.0), 0.0)
    xq = x_ref[0, :, pl.ds(qi * _QBLK, _QBLK)]
    out = (xq - mean) / (jnp.sqrt(var) + 1e-5)
    out_ref[0] = out * alpha_ref[...] + beta_ref[...]


@jax.jit
def kernel(x, xyz, alpha, beta):
    B, C, N = x.shape
    a2 = alpha.reshape(C, 1).astype(jnp.float32)
    b2 = beta.reshape(C, 1).astype(jnp.float32)
    grid = (B, N // _QBLK)
    return pl.pallas_call(
        _ga_block_kernel,
        grid=grid,
        in_specs=[
            pl.BlockSpec((1, 3, N), lambda b, q: (b, 0, 0)),
            pl.BlockSpec((1, C, N), lambda b, q: (b, 0, 0)),
            pl.BlockSpec((C, 1), lambda b, q: (0, 0)),
            pl.BlockSpec((C, 1), lambda b, q: (0, 0)),
        ],
        out_specs=pl.BlockSpec((1, C, _QBLK), lambda b, q: (b, 0, q)),
        out_shape=jax.ShapeDtypeStruct((B, C, N), jnp.float32),
        scratch_shapes=[pltpu.VMEM((_QBLK, N), jnp.float32)],
    )(xyz, x, a2, b2)


# speedup vs baseline: 52.7584x; 2.7184x over previous
import jax
import jax.numpy as jnp
from jax.experimental import pallas as pl
from jax.experimental.pallas import tpu as pltpu

_K = 24
_QBLK = 256
_CHUNK = 128
_TOPG = 4


def _ga_block_kernel(xyz_ref, x_ref, alpha_ref, beta_ref, out_ref, d_ref):
    qi = pl.program_id(1)
    n = x_ref.shape[2]
    c = x_ref.shape[1]
    nch = n // _CHUNK

    keys = xyz_ref[0]
    q = xyz_ref[0, :, pl.ds(qi * _QBLK, _QBLK)]
    ksq = jnp.sum(keys * keys, axis=0, keepdims=True)
    cross = jax.lax.dot_general(
        q, keys, (((0,), (0,)), ((), ())),
        preferred_element_type=jnp.float32)
    d_ref[...] = -2.0 * cross + ksq

    inf = jnp.float32(jnp.inf)
    big = jnp.full((_QBLK, _CHUNK), inf, jnp.float32)
    r1, r2, r3, r4 = big, big, big, big
    for a in range(nch):
        ch = d_ref[:, a * _CHUNK:(a + 1) * _CHUNK]
        carry = jnp.maximum(r1, ch)
        r1 = jnp.minimum(r1, ch)
        new2 = jnp.minimum(r2, carry)
        carry = jnp.maximum(r2, carry)
        r2 = new2
        new3 = jnp.minimum(r3, carry)
        carry = jnp.maximum(r3, carry)
        r3 = new3
        r4 = jnp.minimum(r4, carry)
    cands = jnp.concatenate([r1, r2, r3, r4], axis=1)

    def body(_, cd):
        m = jnp.min(cd, axis=1, keepdims=True)
        return jnp.where(cd == m, inf, cd)

    cd = jax.lax.fori_loop(0, _K - 1, body, cands)
    t = jnp.min(cd, axis=1, keepdims=True)

    sel = (d_ref[...] <= t).astype(jnp.float32)

    x_b = x_ref[0]
    x_hi = x_b.astype(jnp.bfloat16).astype(jnp.float32)
    x_lo = x_b - x_hi
    xsq = x_b * x_b
    q_hi = xsq.astype(jnp.bfloat16).astype(jnp.float32)
    q_lo = xsq - q_hi
    xa = jnp.concatenate(
        [x_hi, x_lo, q_hi, q_lo, jnp.ones((1, n), jnp.float32)], axis=0)
    s = jax.lax.dot_general(
        xa, sel, (((1,), (1,)), ((), ())),
        preferred_element_type=jnp.float32)
    s1 = s[:c] + s[c:2 * c]
    s2 = s[2 * c:3 * c] + s[3 * c:4 * c]
    cnt = s[4 * c:4 * c + 1]
    mean = s1 / cnt
    var = jnp.maximum((s2 - cnt * mean * mean) / (cnt - 1.0), 0.0)
    xq = x_ref[0, :, pl.ds(qi * _QBLK, _QBLK)]
    out = (xq - mean) / (jnp.sqrt(var) + 1e-5)
    out_ref[0] = out * alpha_ref[...] + beta_ref[...]


@jax.jit
def kernel(x, xyz, alpha, beta):
    B, C, N = x.shape
    a2 = alpha.reshape(C, 1).astype(jnp.float32)
    b2 = beta.reshape(C, 1).astype(jnp.float32)
    grid = (B, N // _QBLK)
    return pl.pallas_call(
        _ga_block_kernel,
        grid=grid,
        in_specs=[
            pl.BlockSpec((1, 3, N), lambda b, q: (b, 0, 0)),
            pl.BlockSpec((1, C, N), lambda b, q: (b, 0, 0)),
            pl.BlockSpec((C, 1), lambda b, q: (0, 0)),
            pl.BlockSpec((C, 1), lambda b, q: (0, 0)),
        ],
        out_specs=pl.BlockSpec((1, C, _QBLK), lambda b, q: (b, 0, q)),
        out_shape=jax.ShapeDtypeStruct((B, C, N), jnp.float32),
        scratch_shapes=[pltpu.VMEM((_QBLK, N), jnp.float32)],
    )(xyz, x, a2, b2)


# speedup vs baseline: 59.3346x; 1.1246x over previous
import jax
import jax.numpy as jnp
from jax.experimental import pallas as pl
from jax.experimental.pallas import tpu as pltpu

_K = 24
_QBLK = 256
_CHUNK = 128
_TOPG = 4


def _ga_block_kernel(xyz_ref, x_ref, alpha_ref, beta_ref, out_ref, d_ref):
    qi = pl.program_id(1)
    n = x_ref.shape[2]
    c = x_ref.shape[1]
    nch = n // _CHUNK

    keys = xyz_ref[0]
    q = xyz_ref[0, :, pl.ds(qi * _QBLK, _QBLK)]
    ksq = jnp.sum(keys * keys, axis=0, keepdims=True)
    cross = jax.lax.dot_general(
        q, keys, (((0,), (0,)), ((), ())),
        preferred_element_type=jnp.float32)
    d_ref[...] = -2.0 * cross + ksq

    inf = jnp.float32(jnp.inf)
    big = jnp.full((_QBLK, _CHUNK), inf, jnp.float32)
    r1, r2, r3, r4 = big, big, big, big
    for a in range(nch):
        ch = d_ref[:, a * _CHUNK:(a + 1) * _CHUNK]
        carry = jnp.maximum(r1, ch)
        r1 = jnp.minimum(r1, ch)
        new2 = jnp.minimum(r2, carry)
        carry = jnp.maximum(r2, carry)
        r2 = new2
        new3 = jnp.minimum(r3, carry)
        carry = jnp.maximum(r3, carry)
        r3 = new3
        r4 = jnp.minimum(r4, carry)
    cands = jnp.concatenate([r1, r2, r3, r4], axis=1)

    s = [jnp.full((_QBLK, 32), inf, jnp.float32) for _ in range(8)]
    for a in range(16):
        carry = cands[:, a * 32:(a + 1) * 32]
        for i in range(7):
            new = jnp.minimum(s[i], carry)
            carry = jnp.maximum(s[i], carry)
            s[i] = new
        s[7] = jnp.minimum(s[7], carry)
    cands2 = jnp.concatenate(s, axis=1)

    ct = jnp.transpose(cands2)

    def body(_, cd):
        m = jnp.min(cd, axis=0, keepdims=True)
        return jnp.where(cd == m, inf, cd)

    cd = jax.lax.fori_loop(0, _K - 1, body, ct)
    t_row = jnp.min(cd, axis=0, keepdims=True)
    t = jnp.transpose(t_row)

    sel = (d_ref[...] <= t).astype(jnp.float32)

    x_b = x_ref[0]
    x_hi = x_b.astype(jnp.bfloat16).astype(jnp.float32)
    x_lo = x_b - x_hi
    xsq = x_b * x_b
    q_hi = xsq.astype(jnp.bfloat16).astype(jnp.float32)
    q_lo = xsq - q_hi
    xa = jnp.concatenate(
        [x_hi, x_lo, q_hi, q_lo, jnp.ones((1, n), jnp.float32)], axis=0)
    s = jax.lax.dot_general(
        xa, sel, (((1,), (1,)), ((), ())),
        preferred_element_type=jnp.float32)
    s1 = s[:c] + s[c:2 * c]
    s2 = s[2 * c:3 * c] + s[3 * c:4 * c]
    cnt = s[4 * c:4 * c + 1]
    mean = s1 / cnt
    var = jnp.maximum((s2 - cnt * mean * mean) / (cnt - 1.0), 0.0)
    xq = x_ref[0, :, pl.ds(qi * _QBLK, _QBLK)]
    out = (xq - mean) / (jnp.sqrt(var) + 1e-5)
    out_ref[0] = out * alpha_ref[...] + beta_ref[...]


@jax.jit
def kernel(x, xyz, alpha, beta):
    B, C, N = x.shape
    a2 = alpha.reshape(C, 1).astype(jnp.float32)
    b2 = beta.reshape(C, 1).astype(jnp.float32)
    grid = (B, N // _QBLK)
    return pl.pallas_call(
        _ga_block_kernel,
        grid=grid,
        in_specs=[
            pl.BlockSpec((1, 3, N), lambda b, q: (b, 0, 0)),
            pl.BlockSpec((1, C, N), lambda b, q: (b, 0, 0)),
            pl.BlockSpec((C, 1), lambda b, q: (0, 0)),
            pl.BlockSpec((C, 1), lambda b, q: (0, 0)),
        ],
        out_specs=pl.BlockSpec((1, C, _QBLK), lambda b, q: (b, 0, q)),
        out_shape=jax.ShapeDtypeStruct((B, C, N), jnp.float32),
        scratch_shapes=[pltpu.VMEM((_QBLK, N), jnp.float32)],
    )(xyz, x, a2, b2)


# speedup vs baseline: 85.2098x; 1.4361x over previous
import jax
import jax.numpy as jnp
from jax.experimental import pallas as pl
from jax.experimental.pallas import tpu as pltpu

_K = 24
_QBLK = 256
_CHUNK = 128
_TOPG = 4


def _ga_block_kernel(xyz_ref, x_ref, alpha_ref, beta_ref, out_ref, d_ref):
    qi = pl.program_id(1)
    n = x_ref.shape[2]
    c = x_ref.shape[1]
    nch = n // _CHUNK

    keys = xyz_ref[0]
    q = xyz_ref[0, :, pl.ds(qi * _QBLK, _QBLK)]
    ksq = jnp.sum(keys * keys, axis=0, keepdims=True)
    cross = jax.lax.dot_general(
        q, keys, (((0,), (0,)), ((), ())),
        preferred_element_type=jnp.float32)
    d_ref[...] = -2.0 * cross + ksq

    inf = jnp.float32(jnp.inf)
    big = jnp.full((_QBLK, _CHUNK), inf, jnp.float32)
    r1, r2, r3, r4 = big, big, big, big
    for a in range(nch):
        ch = d_ref[:, a * _CHUNK:(a + 1) * _CHUNK]
        carry = jnp.maximum(r1, ch)
        r1 = jnp.minimum(r1, ch)
        new2 = jnp.minimum(r2, carry)
        carry = jnp.maximum(r2, carry)
        r2 = new2
        new3 = jnp.minimum(r3, carry)
        carry = jnp.maximum(r3, carry)
        r3 = new3
        r4 = jnp.minimum(r4, carry)
    cands = jnp.concatenate([r1, r2, r3, r4], axis=1)

    ct0 = jnp.transpose(cands)

    s = [jnp.full((32, _QBLK), inf, jnp.float32) for _ in range(8)]
    for a in range(16):
        carry = ct0[a * 32:(a + 1) * 32, :]
        for i in range(7):
            new = jnp.minimum(s[i], carry)
            carry = jnp.maximum(s[i], carry)
            s[i] = new
        s[7] = jnp.minimum(s[7], carry)
    ct = jnp.concatenate(s, axis=0)

    for _ in range(_K - 1):
        m = jnp.min(ct, axis=0, keepdims=True)
        ct = jnp.where(ct == m, inf, ct)
    t_row = jnp.min(ct, axis=0, keepdims=True)
    t = jnp.transpose(t_row)

    sel = (d_ref[...] <= t).astype(jnp.float32)

    x_b = x_ref[0]
    x_hi = x_b.astype(jnp.bfloat16).astype(jnp.float32)
    x_lo = x_b - x_hi
    xsq = x_b * x_b
    q_hi = xsq.astype(jnp.bfloat16).astype(jnp.float32)
    q_lo = xsq - q_hi
    xa = jnp.concatenate(
        [x_hi, x_lo, q_hi, q_lo, jnp.ones((1, n), jnp.float32)], axis=0)
    s = jax.lax.dot_general(
        xa, sel, (((1,), (1,)), ((), ())),
        preferred_element_type=jnp.float32)
    s1 = s[:c] + s[c:2 * c]
    s2 = s[2 * c:3 * c] + s[3 * c:4 * c]
    cnt = s[4 * c:4 * c + 1]
    mean = s1 / cnt
    var = jnp.maximum((s2 - cnt * mean * mean) / (cnt - 1.0), 0.0)
    xq = x_ref[0, :, pl.ds(qi * _QBLK, _QBLK)]
    out = (xq - mean) / (jnp.sqrt(var) + 1e-5)
    out_ref[0] = out * alpha_ref[...] + beta_ref[...]


@jax.jit
def kernel(x, xyz, alpha, beta):
    B, C, N = x.shape
    a2 = alpha.reshape(C, 1).astype(jnp.float32)
    b2 = beta.reshape(C, 1).astype(jnp.float32)
    grid = (B, N // _QBLK)
    return pl.pallas_call(
        _ga_block_kernel,
        grid=grid,
        in_specs=[
            pl.BlockSpec((1, 3, N), lambda b, q: (b, 0, 0)),
            pl.BlockSpec((1, C, N), lambda b, q: (b, 0, 0)),
            pl.BlockSpec((C, 1), lambda b, q: (0, 0)),
            pl.BlockSpec((C, 1), lambda b, q: (0, 0)),
        ],
        out_specs=pl.BlockSpec((1, C, _QBLK), lambda b, q: (b, 0, q)),
        out_shape=jax.ShapeDtypeStruct((B, C, N), jnp.float32),
        scratch_shapes=[pltpu.VMEM((_QBLK, N), jnp.float32)],
    )(xyz, x, a2, b2)
